# baseline (speedup 1.0000x reference)
import jax
import jax.numpy as jnp
from jax.experimental import pallas as pl
from jax.experimental.pallas import tpu as pltpu


def _make_csel(W):
    r = jax.lax.broadcasted_iota(jnp.int32, (W - 1, W // 2), 0)
    c = jax.lax.broadcasted_iota(jnp.int32, (W - 1, W // 2), 1)
    return (r == 2 * c).astype(jnp.float32)


def _make_rsel(M):
    r = jax.lax.broadcasted_iota(jnp.int32, (M // 2, M - 1), 0)
    c = jax.lax.broadcasted_iota(jnp.int32, (M // 2, M - 1), 1)
    return (c == 2 * r).astype(jnp.float32)


def _pool2(a, rsel, csel):
    M, W = a.shape
    rowmax = jnp.maximum(a[:M - 1], a[1:])
    rp = jnp.dot(rsel, rowmax, preferred_element_type=jnp.float32)
    m = jnp.maximum(rp[:, :W - 1], rp[:, 1:])
    return jnp.dot(m, csel, preferred_element_type=jnp.float32)


def _fused_cnn(xp3, fcw4, fcb, w1s, b1s, w2s, b2s):
    P, _, H, Wp2 = xp3.shape
    S1 = Wp2 // 2
    W = S1 - 2
    C1 = w1s.shape[0]
    C2 = w2s.shape[0]
    Ho, Wo = H // 2, W // 2
    S2 = Wo + 2
    Ho2, Wo2 = Ho // 2, Wo // 2
    ncls = fcw4.shape[0]
    AW = Wp2 - 2
    AW2 = 2 * S2 - 2
    H1R = Ho + 2
    FB = 128

    def body(xp_ref, fcw_ref, fcb_ref, w1_ref, b1_ref, w2_ref, b2_ref,
             o_ref, h1s, h2s):
        csel1 = _make_csel(AW)
        csel2 = _make_csel(AW2)
        rsel1 = _make_rsel(C1 * 8)
        rsel1r = _make_rsel(C1 * (H % 8)) if H % 8 else None
        rsel2 = _make_rsel(C2 * 8)
        rsel2r = _make_rsel(C2 * (Ho % 8)) if Ho % 8 else None

        h1s[...] = jnp.zeros((3, C1, H1R, 2 * S2), jnp.float32)
        for r0 in range(0, H, 8):
            R = min(8, H - r0)
            accs = [None] * C1
            for dy in range(3):
                for dx in range(3):
                    tv = xp_ref[0, dy, r0:r0 + R, dx:dx + AW]
                    for co in range(C1):
                        t = tv * w1_ref[co, dy * 3 + dx]
                        accs[co] = t if accs[co] is None else accs[co] + t
            stk = jnp.concatenate(accs, axis=0)
            p = _pool2(stk, rsel1 if R == 8 else rsel1r, csel1)
            q = R // 2
            p0 = r0 // 2
            for co in range(C1):
                h = jnp.maximum(p[co * q:co * q + q, :] + b1_ref[co], 0.0)
                ha, hb = h[:, 0:Wo], h[:, Wo + 1:2 * Wo + 1]
                for dy in range(3):
                    lo = p0 + 1 - dy
                    sa, sb = ha, hb
                    if lo < 0:
                        sa, sb = ha[-lo:], hb[-lo:]
                        lo = 0
                    h1s[dy, co, lo:lo + sa.shape[0], 1:1 + Wo] = sa
                    h1s[dy, co, lo:lo + sb.shape[0], S2 + 1:S2 + 1 + Wo] = sb

        for r0 in range(0, Ho, 8):
            R = min(8, Ho - r0)

            def ci_body(ci, accs, r0=r0, R=R):
                out = list(accs)
                for dy in range(3):
                    for dx in range(3):
                        tv = h1s[dy, ci, r0:r0 + R, dx:dx + AW2]
                        k = ci * 9 + dy * 3 + dx
                        for co in range(C2):
                            out[co] = out[co] + tv * w2_ref[co, k]
                return tuple(out)

            zero = jnp.zeros((R, AW2), jnp.float32)
            accs = jax.lax.fori_loop(0, C1, ci_body, (zero,) * C2)
            stk = jnp.concatenate(accs, axis=0)
            p = _pool2(stk, rsel2 if R == 8 else rsel2r, csel2)
            q = R // 2
            for co in range(C2):
                h = jnp.maximum(p[co * q:co * q + q, :] + b2_ref[co], 0.0)
                h2s[co, r0 // 2:r0 // 2 + q, 0:Wo2 + 1] = h[:, 0:Wo2 + 1]
                h2s[co, r0 // 2:r0 // 2 + q, FB:FB + Wo2] = (
                    h[:, Wo2 + 1:2 * Wo2 + 1])

        def fc_body(co, parts):
            pa, pb = parts
            va = h2s[co, :, 0:Wo2]
            vb = h2s[co, :, FB:FB + Wo2]
            na = tuple(
                pa[cls] + jnp.sum(va * fcw_ref[cls, co],
                                  axis=0, keepdims=True)
                for cls in range(ncls))
            nb = tuple(
                pb[cls] + jnp.sum(vb * fcw_ref[cls, co],
                                  axis=0, keepdims=True)
                for cls in range(ncls))
            return (na, nb)

        zp = (jnp.zeros((1, Wo2), jnp.float32),) * ncls
        pa, pb = jax.lax.fori_loop(0, C2, fc_body, (zp, zp))
        iota = jax.lax.broadcasted_iota(jnp.int32, (1, ncls), 1)
        for img, parts in enumerate((pa, pb)):
            r = fcb_ref[0:1, :]
            for cls in range(ncls):
                s = jnp.sum(parts[cls], axis=1, keepdims=True)
                r = r + jnp.where(iota == cls,
                                  jnp.broadcast_to(s, (1, ncls)), 0.0)
            o_ref[img:img + 1] = r.reshape(1, 1, ncls)

    return pl.pallas_call(
        body,
        out_shape=jax.ShapeDtypeStruct((2 * P, 1, ncls), jnp.float32),
        grid=(P,),
        in_specs=[
            pl.BlockSpec((1, 3, H, Wp2), lambda n: (n, 0, 0, 0)),
            pl.BlockSpec((ncls, C2, Ho2, Wo2), lambda n: (0, 0, 0, 0)),
            pl.BlockSpec((1, ncls), lambda n: (0, 0)),
            pl.BlockSpec(memory_space=pltpu.SMEM),
            pl.BlockSpec(memory_space=pltpu.SMEM),
            pl.BlockSpec(memory_space=pltpu.SMEM),
            pl.BlockSpec(memory_space=pltpu.SMEM),
        ],
        out_specs=pl.BlockSpec((2, 1, ncls), lambda n: (n, 0, 0)),
        scratch_shapes=[
            pltpu.VMEM((3, C1, Ho + 2, 2 * S2), jnp.float32),
            pltpu.VMEM((C2, Ho2, FB + Wo2), jnp.float32),
        ],
        compiler_params=pltpu.CompilerParams(
            dimension_semantics=("parallel",)),
    )(xp3, fcw4, fcb, w1s, b1s, w2s, b2s)


@jax.jit
def _forward(x_nchw, w1, b1, w2, b2, fcw, fcb):
    N, Cin, H, W = x_nchw.shape
    C1 = w1.shape[0]
    C2 = w2.shape[0]
    Ho2, Wo2 = H // 4, W // 4
    ncls = fcb.shape[-1]
    xpair = jnp.pad(
        x_nchw.astype(jnp.float32).reshape(N // 2, 2, H, W),
        ((0, 0), (0, 0), (1, 1), (1, 1)))
    xp = xpair.transpose(0, 2, 1, 3).reshape(N // 2, H + 2, 2 * (W + 2))
    xp3 = jnp.stack([xp[:, dy:dy + H, :] for dy in range(3)], axis=1)
    w1s = w1.reshape(C1, Cin * 9).astype(jnp.float32)
    w2s = w2.reshape(C2, C1 * 9).astype(jnp.float32)
    b1s = b1.reshape(C1).astype(jnp.float32)
    b2s = b2.reshape(C2).astype(jnp.float32)
    K2 = C2 * Ho2 * Wo2
    fcw4 = fcw[:, :K2].reshape(ncls, C2, Ho2, Wo2).astype(jnp.float32)
    fcbr = fcb.reshape(1, ncls).astype(jnp.float32)
    out = _fused_cnn(xp3, fcw4, fcbr, w1s, b1s, w2s, b2s)
    return out.reshape(N, ncls)


def kernel(x_nchw, w1, b1, w2, b2, fcw, fcb):
    return _forward(x_nchw, w1, b1, w2, b2, fcw, fcb)


# speedup vs baseline: 1.0489x; 1.0489x over previous
import jax
import jax.numpy as jnp
from jax.experimental import pallas as pl
from jax.experimental.pallas import tpu as pltpu


def _make_csel(W):
    r = jax.lax.broadcasted_iota(jnp.int32, (W - 1, W // 2), 0)
    c = jax.lax.broadcasted_iota(jnp.int32, (W - 1, W // 2), 1)
    return (r == 2 * c).astype(jnp.float32)


def _make_rsel(M, rows):
    r = jax.lax.broadcasted_iota(jnp.int32, (rows, M), 0)
    c = jax.lax.broadcasted_iota(jnp.int32, (rows, M), 1)
    return (c == 2 * r).astype(jnp.float32)


def _pool2(a, rsel_ref, csel_ref):
    M, W = a.shape
    rowmax = jnp.maximum(a[:M - 1], a[1:])
    rsel = rsel_ref[0:M // 2, 0:M - 1]
    rp = jnp.dot(rsel, rowmax, preferred_element_type=jnp.float32)
    m = jnp.maximum(rp[:, :W - 1], rp[:, 1:])
    return jnp.dot(m, csel_ref[...], preferred_element_type=jnp.float32)


def _fused_cnn(xp3, fcw4, fcb, csel1, csel2, rsel1, rsel2, w1s, b1s,
               w2s, b2s):
    P, _, H, Wp2 = xp3.shape
    S1 = Wp2 // 2
    W = S1 - 2
    C1 = w1s.shape[0]
    C2 = w2s.shape[0]
    Ho, Wo = H // 2, W // 2
    S2 = Wo + 2
    Ho2, Wo2 = Ho // 2, Wo // 2
    ncls = fcw4.shape[0]
    AW = Wp2 - 2
    AW2 = 2 * S2 - 2
    FB = 128

    def body(xp_ref, fcw_ref, fcb_ref, cs1_ref, cs2_ref, rs1_ref, rs2_ref,
             w1_ref, b1_ref, w2_ref, b2_ref, o_ref, h1s, h2s):
        h1s[...] = jnp.zeros((3, C1, Ho + 2, 2 * S2), jnp.float32)
        for r0 in range(0, H, 8):
            R = min(8, H - r0)
            accs = [None] * C1
            for dy in range(3):
                for dx in range(3):
                    tv = xp_ref[0, dy, r0:r0 + R, dx:dx + AW]
                    for co in range(C1):
                        t = tv * w1_ref[co, dy * 3 + dx]
                        accs[co] = t if accs[co] is None else accs[co] + t
            stk = jnp.concatenate(accs, axis=0)
            p = _pool2(stk, rs1_ref, cs1_ref)
            q = R // 2
            p0 = r0 // 2
            for co in range(C1):
                h = jnp.maximum(p[co * q:co * q + q, :] + b1_ref[co], 0.0)
                ha, hb = h[:, 0:Wo], h[:, Wo + 1:2 * Wo + 1]
                for dy in range(3):
                    lo = p0 + 1 - dy
                    sa, sb = ha, hb
                    if lo < 0:
                        sa, sb = ha[-lo:], hb[-lo:]
                        lo = 0
                    h1s[dy, co, lo:lo + sa.shape[0], 1:1 + Wo] = sa
                    h1s[dy, co, lo:lo + sb.shape[0], S2 + 1:S2 + 1 + Wo] = sb

        for r0 in range(0, Ho, 8):
            R = min(8, Ho - r0)

            def ci_body(cih, accs, r0=r0, R=R):
                out = list(accs)
                for sub in range(2):
                    ci = 2 * cih + sub
                    for dy in range(3):
                        for dx in range(3):
                            tv = h1s[dy, ci, r0:r0 + R, dx:dx + AW2]
                            k = ci * 9 + dy * 3 + dx
                            for co in range(C2):
                                out[co] = out[co] + tv * w2_ref[co, k]
                return tuple(out)

            zero = jnp.zeros((R, AW2), jnp.float32)
            accs = jax.lax.fori_loop(0, C1 // 2, ci_body, (zero,) * C2)
            stk = jnp.concatenate(accs, axis=0)
            p = _pool2(stk, rs2_ref, cs2_ref)
            q = R // 2
            for co in range(C2):
                h = jnp.maximum(p[co * q:co * q + q, :] + b2_ref[co], 0.0)
                h2s[co, r0 // 2:r0 // 2 + q, 0:Wo2 + 1] = h[:, 0:Wo2 + 1]
                h2s[co, r0 // 2:r0 // 2 + q, FB:FB + Wo2] = (
                    h[:, Wo2 + 1:2 * Wo2 + 1])

        def fc_body(co, parts):
            pa, pb = parts
            va = h2s[co, :, 0:Wo2]
            vb = h2s[co, :, FB:FB + Wo2]
            na = tuple(
                pa[cls] + jnp.sum(va * fcw_ref[cls, co],
                                  axis=0, keepdims=True)
                for cls in range(ncls))
            nb = tuple(
                pb[cls] + jnp.sum(vb * fcw_ref[cls, co],
                                  axis=0, keepdims=True)
                for cls in range(ncls))
            return (na, nb)

        zp = (jnp.zeros((1, Wo2), jnp.float32),) * ncls
        pa, pb = jax.lax.fori_loop(0, C2, fc_body, (zp, zp))
        iota = jax.lax.broadcasted_iota(jnp.int32, (1, ncls), 1)
        for img, parts in enumerate((pa, pb)):
            r = fcb_ref[0:1, :]
            for cls in range(ncls):
                s = jnp.sum(parts[cls], axis=1, keepdims=True)
                r = r + jnp.where(iota == cls,
                                  jnp.broadcast_to(s, (1, ncls)), 0.0)
            o_ref[img:img + 1] = r.reshape(1, 1, ncls)

    return pl.pallas_call(
        body,
        out_shape=jax.ShapeDtypeStruct((2 * P, 1, ncls), jnp.float32),
        grid=(P,),
        in_specs=[
            pl.BlockSpec((1, 3, H, Wp2), lambda n: (n, 0, 0, 0)),
            pl.BlockSpec((ncls, C2, Ho2, Wo2), lambda n: (0, 0, 0, 0)),
            pl.BlockSpec((1, ncls), lambda n: (0, 0)),
            pl.BlockSpec(csel1.shape, lambda n: (0, 0)),
            pl.BlockSpec(csel2.shape, lambda n: (0, 0)),
            pl.BlockSpec(rsel1.shape, lambda n: (0, 0)),
            pl.BlockSpec(rsel2.shape, lambda n: (0, 0)),
            pl.BlockSpec(memory_space=pltpu.SMEM),
            pl.BlockSpec(memory_space=pltpu.SMEM),
            pl.BlockSpec(memory_space=pltpu.SMEM),
            pl.BlockSpec(memory_space=pltpu.SMEM),
        ],
        out_specs=pl.BlockSpec((2, 1, ncls), lambda n: (n, 0, 0)),
        scratch_shapes=[
            pltpu.VMEM((3, C1, Ho + 2, 2 * S2), jnp.float32),
            pltpu.VMEM((C2, Ho2, FB + Wo2), jnp.float32),
        ],
        compiler_params=pltpu.CompilerParams(
            dimension_semantics=("parallel",)),
    )(xp3, fcw4, fcb, csel1, csel2, rsel1, rsel2, w1s, b1s, w2s, b2s)


@jax.jit
def _forward(x_nchw, w1, b1, w2, b2, fcw, fcb):
    N, Cin, H, W = x_nchw.shape
    C1 = w1.shape[0]
    C2 = w2.shape[0]
    Ho2, Wo2 = H // 4, W // 4
    ncls = fcb.shape[-1]
    xpair = jnp.pad(
        x_nchw.astype(jnp.float32).reshape(N // 2, 2, H, W),
        ((0, 0), (0, 0), (1, 1), (1, 1)))
    xp = xpair.transpose(0, 2, 1, 3).reshape(N // 2, H + 2, 2 * (W + 2))
    xp3 = jnp.stack([xp[:, dy:dy + H, :] for dy in range(3)], axis=1)
    AW = 2 * (W + 2) - 2
    AW2 = 2 * (W // 2 + 2) - 2
    csel1 = _make_csel(AW)
    csel2 = _make_csel(AW2)
    rsel1 = _make_rsel(C1 * 8 - 1, C1 * 4)
    rsel2 = _make_rsel(C2 * 8 - 1, C2 * 4)
    w1s = w1.reshape(C1, Cin * 9).astype(jnp.float32)
    w2s = w2.reshape(C2, C1 * 9).astype(jnp.float32)
    b1s = b1.reshape(C1).astype(jnp.float32)
    b2s = b2.reshape(C2).astype(jnp.float32)
    K2 = C2 * Ho2 * Wo2
    fcw4 = fcw[:, :K2].reshape(ncls, C2, Ho2, Wo2).astype(jnp.float32)
    fcbr = fcb.reshape(1, ncls).astype(jnp.float32)
    out = _fused_cnn(xp3, fcw4, fcbr, csel1, csel2, rsel1, rsel2,
                     w1s, b1s, w2s, b2s)
    return out.reshape(N, ncls)


def kernel(x_nchw, w1, b1, w2, b2, fcw, fcb):
    return _forward(x_nchw, w1, b1, w2, b2, fcw, fcb)
